# baseline (speedup 1.0000x reference)
import functools

import jax
import jax.numpy as jnp
from jax import lax
from jax.experimental import pallas as pl
from jax.experimental.pallas import tpu as pltpu
from jax.experimental.pallas import tpu_sc as plsc

N_NEURONS = 1000
EMBED_DIM = 128
BATCH = 1024
N_SELECTED = 16384

NC, NS, L = 2, 16, 16
NW = NC * NS
B_PER_W = N_SELECTED // NW
AUX_W = 128
AUX_OUT = 16


def _worker_base():
    wid = lax.axis_index("s") * NC + lax.axis_index("c")
    return wid * B_PER_W


def _sc_aux_body(aux_hbm, keys_hbm, aux_g_hbm, idx_v, rows_v, pack_v, sem):
    base = _worker_base()
    pltpu.sync_copy(keys_hbm.at[pl.ds(base, B_PER_W)], idx_v)
    pltpu.async_copy(aux_hbm.at[idx_v], rows_v, sem).wait()

    def body(r, _):
        pack_v[pl.ds(r * AUX_OUT, AUX_OUT)] = rows_v[r, pl.ds(0, AUX_OUT)]
        return 0

    lax.fori_loop(0, B_PER_W, body, 0)
    pltpu.sync_copy(pack_v,
                    aux_g_hbm.at[pl.ds(base * AUX_OUT, B_PER_W * AUX_OUT)])


def _sc_emb_body(table_hbm, keys_hbm, emb_hbm, idx_v, rows_v, sem):
    base = _worker_base()
    pltpu.sync_copy(keys_hbm.at[pl.ds(base, B_PER_W)], idx_v)
    pltpu.async_copy(table_hbm.at[idx_v], rows_v, sem).wait()
    pltpu.sync_copy(rows_v, emb_hbm.at[pl.ds(base, B_PER_W)])


def _sc_mesh():
    return plsc.VectorSubcoreMesh(core_axis_name="c", subcore_axis_name="s",
                                  num_cores=NC, num_subcores=NS)


@functools.cache
def _sc_aux():
    return pl.kernel(
        _sc_aux_body,
        out_type=jax.ShapeDtypeStruct((N_SELECTED * AUX_OUT,), jnp.float32),
        mesh=_sc_mesh(),
        scratch_types=[
            pltpu.VMEM((B_PER_W,), jnp.int32),
            pltpu.VMEM((B_PER_W, AUX_W), jnp.float32),
            pltpu.VMEM((B_PER_W * AUX_OUT,), jnp.float32),
            pltpu.SemaphoreType.DMA,
        ],
    )


@functools.cache
def _sc_emb():
    return pl.kernel(
        _sc_emb_body,
        out_type=jax.ShapeDtypeStruct((N_SELECTED, EMBED_DIM), jnp.float32),
        mesh=_sc_mesh(),
        scratch_types=[
            pltpu.VMEM((B_PER_W,), jnp.int32),
            pltpu.VMEM((B_PER_W, EMBED_DIM), jnp.float32),
            pltpu.SemaphoreType.DMA,
        ],
    )


def _tc_affine_body(x_ref, s_ref, b_ref, o_ref):
    o_ref[...] = b_ref[...] + s_ref[...] * x_ref[...]


ROW_BLK = 512
COL_BLK = 4096

_tc_affine = pl.pallas_call(
    _tc_affine_body,
    grid=(BATCH // ROW_BLK, N_SELECTED // COL_BLK),
    in_specs=[
        pl.BlockSpec((ROW_BLK, COL_BLK), lambda i, j: (i, j)),
        pl.BlockSpec((1, COL_BLK), lambda i, j: (0, j)),
        pl.BlockSpec((1, COL_BLK), lambda i, j: (0, j)),
    ],
    out_specs=pl.BlockSpec((ROW_BLK, COL_BLK), lambda i, j: (i, j)),
    out_shape=jax.ShapeDtypeStruct((BATCH, N_SELECTED), jnp.float32),
)


def kernel(inputs, input_axon_embeddings, scale, bias, keys_idx):
    keys32 = keys_idx.astype(jnp.int32)
    aux = jnp.concatenate(
        [scale[:, None], bias[:, None],
         jnp.zeros((N_NEURONS, AUX_W - 2), jnp.float32)], axis=1)
    aux_g = _sc_aux()(aux, keys32).reshape(N_SELECTED, AUX_OUT)
    out_emb = _sc_emb()(input_axon_embeddings, keys32)
    scale_g = aux_g[:, 0].reshape(1, N_SELECTED)
    bias_g = aux_g[:, 1].reshape(1, N_SELECTED)
    out_inputs = _tc_affine(inputs, scale_g, bias_g)
    return (out_inputs, out_emb)


# speedup vs baseline: 1.1620x; 1.1620x over previous
import functools

import jax
import jax.numpy as jnp
from jax import lax
from jax.experimental import pallas as pl
from jax.experimental.pallas import tpu as pltpu
from jax.experimental.pallas import tpu_sc as plsc

N_NEURONS = 1000
EMBED_DIM = 128
BATCH = 1024
N_SELECTED = 16384

NC, NS, L = 2, 16, 16
NW = NC * NS
B_PER_W = N_SELECTED // NW
AUX_W = 128


def _worker_base():
    wid = lax.axis_index("s") * NC + lax.axis_index("c")
    return wid * B_PER_W


def _sc_aux_body(aux_hbm, keys_hbm, aux_g_hbm, idx_v, rows_v, sem):
    base = _worker_base()
    pltpu.sync_copy(keys_hbm.at[pl.ds(base, B_PER_W)], idx_v)
    pltpu.async_copy(aux_hbm.at[idx_v], rows_v, sem).wait()
    pltpu.sync_copy(rows_v, aux_g_hbm.at[pl.ds(base, B_PER_W)])


def _sc_emb_body(table_hbm, keys_hbm, emb_hbm, idx_v, rows_v, sem):
    base = _worker_base()
    pltpu.sync_copy(keys_hbm.at[pl.ds(base, B_PER_W)], idx_v)
    pltpu.async_copy(table_hbm.at[idx_v], rows_v, sem).wait()
    pltpu.sync_copy(rows_v, emb_hbm.at[pl.ds(base, B_PER_W)])


def _sc_mesh():
    return plsc.VectorSubcoreMesh(core_axis_name="c", subcore_axis_name="s",
                                  num_cores=NC, num_subcores=NS)


@functools.cache
def _sc_aux():
    return pl.kernel(
        _sc_aux_body,
        out_type=jax.ShapeDtypeStruct((N_SELECTED, AUX_W), jnp.float32),
        mesh=_sc_mesh(),
        scratch_types=[
            pltpu.VMEM((B_PER_W,), jnp.int32),
            pltpu.VMEM((B_PER_W, AUX_W), jnp.float32),
            pltpu.SemaphoreType.DMA,
        ],
    )


@functools.cache
def _sc_emb():
    return pl.kernel(
        _sc_emb_body,
        out_type=jax.ShapeDtypeStruct((N_SELECTED, EMBED_DIM), jnp.float32),
        mesh=_sc_mesh(),
        scratch_types=[
            pltpu.VMEM((B_PER_W,), jnp.int32),
            pltpu.VMEM((B_PER_W, EMBED_DIM), jnp.float32),
            pltpu.SemaphoreType.DMA,
        ],
    )


ROW_BLK = 512
COL_BLK = 4096


def _tc_affine_body(x_ref, aux_ref, o_ref, coef_ref):
    @pl.when(pl.program_id(1) == 0)
    def _():
        lane = lax.broadcasted_iota(jnp.int32, (2, AUX_W), 1)
        row = lax.broadcasted_iota(jnp.int32, (2, AUX_W), 0)
        sel = jnp.where(lane == row, 1.0, 0.0)
        coef_ref[...] = lax.dot_general(
            sel, aux_ref[...], (((1,), (1,)), ((), ())),
            preferred_element_type=jnp.float32,
            precision=lax.Precision.HIGHEST)

    o_ref[...] = coef_ref[1:2, :] + coef_ref[0:1, :] * x_ref[...]


_tc_affine = pl.pallas_call(
    _tc_affine_body,
    grid=(N_SELECTED // COL_BLK, BATCH // ROW_BLK),
    in_specs=[
        pl.BlockSpec((ROW_BLK, COL_BLK), lambda j, i: (i, j)),
        pl.BlockSpec((COL_BLK, AUX_W), lambda j, i: (j, 0)),
    ],
    out_specs=pl.BlockSpec((ROW_BLK, COL_BLK), lambda j, i: (i, j)),
    out_shape=jax.ShapeDtypeStruct((BATCH, N_SELECTED), jnp.float32),
    scratch_shapes=[pltpu.VMEM((2, COL_BLK), jnp.float32)],
)


def kernel(inputs, input_axon_embeddings, scale, bias, keys_idx):
    keys32 = keys_idx.astype(jnp.int32)
    aux = jnp.concatenate(
        [scale[:, None], bias[:, None],
         jnp.zeros((N_NEURONS, AUX_W - 2), jnp.float32)], axis=1)
    aux_g = _sc_aux()(aux, keys32)
    out_emb = _sc_emb()(input_axon_embeddings, keys32)
    out_inputs = _tc_affine(inputs, aux_g)
    return (out_inputs, out_emb)


# speedup vs baseline: 1.1652x; 1.0028x over previous
import functools

import jax
import jax.numpy as jnp
from jax import lax
from jax.experimental import pallas as pl
from jax.experimental.pallas import tpu as pltpu
from jax.experimental.pallas import tpu_sc as plsc

N_NEURONS = 1000
EMBED_DIM = 128
BATCH = 1024
N_SELECTED = 16384

NC, NS, L = 2, 16, 16
NW = NC * NS
B_PER_W = N_SELECTED // NW
AUX_W = 128


def _worker_base():
    wid = lax.axis_index("s") * NC + lax.axis_index("c")
    return wid * B_PER_W


HALF = B_PER_W // 2


def _sc_aux_body(aux_hbm, keys_hbm, aux_g_hbm, idx_v, r0, r1, sem0, sem1):
    base = _worker_base()
    pltpu.sync_copy(keys_hbm.at[pl.ds(base, B_PER_W)], idx_v)
    cp0 = pltpu.async_copy(aux_hbm.at[idx_v.at[pl.ds(0, HALF)]], r0, sem0)
    cp1 = pltpu.async_copy(aux_hbm.at[idx_v.at[pl.ds(HALF, HALF)]], r1, sem1)
    cp0.wait()
    pltpu.sync_copy(r0, aux_g_hbm.at[pl.ds(base, HALF)])
    cp1.wait()
    pltpu.sync_copy(r1, aux_g_hbm.at[pl.ds(base + HALF, HALF)])


def _sc_emb_body(table_hbm, keys_hbm, emb_hbm, idx_v, rows_v, sem):
    base = _worker_base()
    pltpu.sync_copy(keys_hbm.at[pl.ds(base, B_PER_W)], idx_v)
    pltpu.async_copy(table_hbm.at[idx_v], rows_v, sem).wait()
    pltpu.sync_copy(rows_v, emb_hbm.at[pl.ds(base, B_PER_W)])


def _sc_mesh():
    return plsc.VectorSubcoreMesh(core_axis_name="c", subcore_axis_name="s",
                                  num_cores=NC, num_subcores=NS)


@functools.cache
def _sc_aux():
    return pl.kernel(
        _sc_aux_body,
        out_type=jax.ShapeDtypeStruct((N_SELECTED, AUX_W), jnp.float32),
        mesh=_sc_mesh(),
        scratch_types=[
            pltpu.VMEM((B_PER_W,), jnp.int32),
            pltpu.VMEM((HALF, AUX_W), jnp.float32),
            pltpu.VMEM((HALF, AUX_W), jnp.float32),
            pltpu.SemaphoreType.DMA,
            pltpu.SemaphoreType.DMA,
        ],
    )


@functools.cache
def _sc_emb():
    return pl.kernel(
        _sc_emb_body,
        out_type=jax.ShapeDtypeStruct((N_SELECTED, EMBED_DIM), jnp.float32),
        mesh=_sc_mesh(),
        scratch_types=[
            pltpu.VMEM((B_PER_W,), jnp.int32),
            pltpu.VMEM((B_PER_W, EMBED_DIM), jnp.float32),
            pltpu.SemaphoreType.DMA,
        ],
    )


ROW_BLK = 512
COL_BLK = 4096


def _tc_affine_body(x_ref, aux_ref, o_ref, coef_ref):
    @pl.when(pl.program_id(1) == 0)
    def _():
        lane = lax.broadcasted_iota(jnp.int32, (2, AUX_W), 1)
        row = lax.broadcasted_iota(jnp.int32, (2, AUX_W), 0)
        sel = jnp.where(lane == row, 1.0, 0.0)
        coef_ref[...] = lax.dot_general(
            sel, aux_ref[...], (((1,), (1,)), ((), ())),
            preferred_element_type=jnp.float32,
            precision=lax.Precision.DEFAULT)

    o_ref[...] = coef_ref[1:2, :] + coef_ref[0:1, :] * x_ref[...]


_tc_affine = pl.pallas_call(
    _tc_affine_body,
    grid=(N_SELECTED // COL_BLK, BATCH // ROW_BLK),
    in_specs=[
        pl.BlockSpec((ROW_BLK, COL_BLK), lambda j, i: (i, j)),
        pl.BlockSpec((COL_BLK, AUX_W), lambda j, i: (j, 0)),
    ],
    out_specs=pl.BlockSpec((ROW_BLK, COL_BLK), lambda j, i: (i, j)),
    out_shape=jax.ShapeDtypeStruct((BATCH, N_SELECTED), jnp.float32),
    scratch_shapes=[pltpu.VMEM((2, COL_BLK), jnp.float32)],
)


def kernel(inputs, input_axon_embeddings, scale, bias, keys_idx):
    keys32 = keys_idx.astype(jnp.int32)
    aux = jnp.concatenate(
        [scale[:, None], bias[:, None],
         jnp.zeros((N_NEURONS, AUX_W - 2), jnp.float32)], axis=1)
    aux_g = _sc_aux()(aux, keys32)
    out_emb = _sc_emb()(input_axon_embeddings, keys32)
    out_inputs = _tc_affine(inputs, aux_g)
    return (out_inputs, out_emb)


# speedup vs baseline: 1.1862x; 1.0180x over previous
import functools

import jax
import jax.numpy as jnp
from jax import lax
from jax.experimental import pallas as pl
from jax.experimental.pallas import tpu as pltpu
from jax.experimental.pallas import tpu_sc as plsc

N_NEURONS = 1000
EMBED_DIM = 128
BATCH = 1024
N_SELECTED = 16384

NC, NS, L = 2, 16, 16
NW = NC * NS
B_PER_W = N_SELECTED // NW
AUX_F = 32
AUX_B = AUX_F * 4
HALF = B_PER_W // 2


def _worker_base():
    wid = lax.axis_index("s") * NC + lax.axis_index("c")
    return wid * B_PER_W


def _sc_aux_body(scale_hbm, bias_hbm, keys_hbm, sg_hbm, bg_hbm,
                 idx_v, s_v, b_v, sem0, sem1):
    base = _worker_base()
    pltpu.sync_copy(keys_hbm.at[pl.ds(base, B_PER_W)], idx_v)
    cp0 = pltpu.async_copy(scale_hbm.at[idx_v], s_v, sem0)
    cp1 = pltpu.async_copy(bias_hbm.at[idx_v], b_v, sem1)
    cp0.wait()
    pltpu.sync_copy(s_v, sg_hbm.at[pl.ds(base, B_PER_W)])
    cp1.wait()
    pltpu.sync_copy(b_v, bg_hbm.at[pl.ds(base, B_PER_W)])


def _sc_emb_body(table_hbm, keys_hbm, emb_hbm, idx_v, rows_v, sem):
    base = _worker_base()
    pltpu.sync_copy(keys_hbm.at[pl.ds(base, B_PER_W)], idx_v)
    pltpu.async_copy(table_hbm.at[idx_v], rows_v, sem).wait()
    pltpu.sync_copy(rows_v, emb_hbm.at[pl.ds(base, B_PER_W)])


def _sc_mesh():
    return plsc.VectorSubcoreMesh(core_axis_name="c", subcore_axis_name="s",
                                  num_cores=NC, num_subcores=NS)


@functools.cache
def _sc_aux():
    return pl.kernel(
        _sc_aux_body,
        out_type=(
            jax.ShapeDtypeStruct((N_SELECTED,), jnp.float32),
            jax.ShapeDtypeStruct((N_SELECTED,), jnp.float32),
        ),
        mesh=_sc_mesh(),
        scratch_types=[
            pltpu.VMEM((B_PER_W,), jnp.int32),
            pltpu.VMEM((B_PER_W,), jnp.float32),
            pltpu.VMEM((B_PER_W,), jnp.float32),
            pltpu.SemaphoreType.DMA,
            pltpu.SemaphoreType.DMA,
        ],
    )


@functools.cache
def _sc_emb():
    return pl.kernel(
        _sc_emb_body,
        out_type=jax.ShapeDtypeStruct((N_SELECTED, EMBED_DIM), jnp.float32),
        mesh=_sc_mesh(),
        scratch_types=[
            pltpu.VMEM((B_PER_W,), jnp.int32),
            pltpu.VMEM((B_PER_W, EMBED_DIM), jnp.float32),
            pltpu.SemaphoreType.DMA,
        ],
    )


def _tc_affine_body(x_ref, s_ref, b_ref, o_ref):
    o_ref[...] = b_ref[...] + s_ref[...] * x_ref[...]


ROW_BLK = 512
COL_BLK = 4096

_tc_affine = pl.pallas_call(
    _tc_affine_body,
    grid=(BATCH // ROW_BLK, N_SELECTED // COL_BLK),
    in_specs=[
        pl.BlockSpec((ROW_BLK, COL_BLK), lambda i, j: (i, j)),
        pl.BlockSpec((1, COL_BLK), lambda i, j: (0, j)),
        pl.BlockSpec((1, COL_BLK), lambda i, j: (0, j)),
    ],
    out_specs=pl.BlockSpec((ROW_BLK, COL_BLK), lambda i, j: (i, j)),
    out_shape=jax.ShapeDtypeStruct((BATCH, N_SELECTED), jnp.float32),
)


def kernel(inputs, input_axon_embeddings, scale, bias, keys_idx):
    keys32 = keys_idx.astype(jnp.int32)
    sg, bg = _sc_aux()(scale, bias, keys32)
    out_emb = _sc_emb()(input_axon_embeddings, keys32)
    scale_g = sg.reshape(1, N_SELECTED)
    bias_g = bg.reshape(1, N_SELECTED)
    out_inputs = _tc_affine(inputs, scale_g, bias_g)
    return (out_inputs, out_emb)


# speedup vs baseline: 1.1865x; 1.0002x over previous
import functools

import jax
import jax.numpy as jnp
from jax import lax
from jax.experimental import pallas as pl
from jax.experimental.pallas import tpu as pltpu
from jax.experimental.pallas import tpu_sc as plsc

N_NEURONS = 1000
EMBED_DIM = 128
BATCH = 1024
N_SELECTED = 16384

NC, NS, L = 2, 16, 16
NW = NC * NS
B_PER_W = N_SELECTED // NW
AUX_F = 32
AUX_B = AUX_F * 4
HALF = B_PER_W // 2


def _worker_base():
    wid = lax.axis_index("s") * NC + lax.axis_index("c")
    return wid * B_PER_W


NSTREAM = 4
SCHUNK = B_PER_W // NSTREAM


def _sc_aux_body(scale_hbm, bias_hbm, keys_hbm, sg_hbm, bg_hbm,
                 idx_v, s_v, b_v, *sems):
    base = _worker_base()
    pltpu.sync_copy(keys_hbm.at[pl.ds(base, B_PER_W)], idx_v)
    cps = []
    for c in range(NSTREAM):
        sl = pl.ds(c * SCHUNK, SCHUNK)
        cps.append(pltpu.async_copy(
            scale_hbm.at[idx_v.at[sl]], s_v.at[sl], sems[2 * c]))
        cps.append(pltpu.async_copy(
            bias_hbm.at[idx_v.at[sl]], b_v.at[sl], sems[2 * c + 1]))
    for cp in cps:
        cp.wait()
    pltpu.sync_copy(s_v, sg_hbm.at[pl.ds(base, B_PER_W)])
    pltpu.sync_copy(b_v, bg_hbm.at[pl.ds(base, B_PER_W)])


def _sc_emb_body(table_hbm, keys_hbm, emb_hbm, idx_v, rows_v, sem):
    base = _worker_base()
    pltpu.sync_copy(keys_hbm.at[pl.ds(base, B_PER_W)], idx_v)
    pltpu.async_copy(table_hbm.at[idx_v], rows_v, sem).wait()
    pltpu.sync_copy(rows_v, emb_hbm.at[pl.ds(base, B_PER_W)])


def _sc_mesh():
    return plsc.VectorSubcoreMesh(core_axis_name="c", subcore_axis_name="s",
                                  num_cores=NC, num_subcores=NS)


@functools.cache
def _sc_aux():
    return pl.kernel(
        _sc_aux_body,
        out_type=(
            jax.ShapeDtypeStruct((N_SELECTED,), jnp.float32),
            jax.ShapeDtypeStruct((N_SELECTED,), jnp.float32),
        ),
        mesh=_sc_mesh(),
        scratch_types=[
            pltpu.VMEM((B_PER_W,), jnp.int32),
            pltpu.VMEM((B_PER_W,), jnp.float32),
            pltpu.VMEM((B_PER_W,), jnp.float32),
        ] + [pltpu.SemaphoreType.DMA] * (2 * NSTREAM),
    )


@functools.cache
def _sc_emb():
    return pl.kernel(
        _sc_emb_body,
        out_type=jax.ShapeDtypeStruct((N_SELECTED, EMBED_DIM), jnp.float32),
        mesh=_sc_mesh(),
        scratch_types=[
            pltpu.VMEM((B_PER_W,), jnp.int32),
            pltpu.VMEM((B_PER_W, EMBED_DIM), jnp.float32),
            pltpu.SemaphoreType.DMA,
        ],
    )


def _tc_affine_body(x_ref, s_ref, b_ref, o_ref):
    o_ref[...] = b_ref[...] + s_ref[...] * x_ref[...]


ROW_BLK = 512
COL_BLK = 4096

_tc_affine = pl.pallas_call(
    _tc_affine_body,
    grid=(BATCH // ROW_BLK, N_SELECTED // COL_BLK),
    in_specs=[
        pl.BlockSpec((ROW_BLK, COL_BLK), lambda i, j: (i, j)),
        pl.BlockSpec((1, COL_BLK), lambda i, j: (0, j)),
        pl.BlockSpec((1, COL_BLK), lambda i, j: (0, j)),
    ],
    out_specs=pl.BlockSpec((ROW_BLK, COL_BLK), lambda i, j: (i, j)),
    out_shape=jax.ShapeDtypeStruct((BATCH, N_SELECTED), jnp.float32),
)


def kernel(inputs, input_axon_embeddings, scale, bias, keys_idx):
    keys32 = keys_idx.astype(jnp.int32)
    sg, bg = _sc_aux()(scale, bias, keys32)
    out_emb = _sc_emb()(input_axon_embeddings, keys32)
    scale_g = sg.reshape(1, N_SELECTED)
    bias_g = bg.reshape(1, N_SELECTED)
    out_inputs = _tc_affine(inputs, scale_g, bias_g)
    return (out_inputs, out_emb)


# speedup vs baseline: 1.3158x; 1.1090x over previous
import functools

import jax
import jax.numpy as jnp
from jax import lax
from jax.experimental import pallas as pl
from jax.experimental.pallas import tpu as pltpu
from jax.experimental.pallas import tpu_sc as plsc

N_NEURONS = 1000
EMBED_DIM = 128
BATCH = 1024
N_SELECTED = 16384

NC, NS, L = 2, 16, 16
NW = NC * NS
B_PER_W = N_SELECTED // NW
AUX_F = 32
AUX_B = AUX_F * 4
HALF = B_PER_W // 2


def _worker_base():
    wid = lax.axis_index("s") * NC + lax.axis_index("c")
    return wid * B_PER_W


NSTREAM = 4
SCHUNK = B_PER_W // NSTREAM


def _sc_aux_body(scale_hbm, bias_hbm, keys_hbm, sg_hbm, bg_hbm,
                 idx_v, s_v, b_v, s_sh, b_sh, sem0, sem1):
    base = _worker_base()

    @pl.when(lax.axis_index("s") == 0)
    def _():
        pltpu.sync_copy(scale_hbm, s_sh)
        pltpu.sync_copy(bias_hbm, b_sh)

    pltpu.sync_copy(keys_hbm.at[pl.ds(base, B_PER_W)], idx_v)
    plsc.subcore_barrier()
    cp0 = pltpu.async_copy(s_sh.at[idx_v], s_v, sem0)
    cp1 = pltpu.async_copy(b_sh.at[idx_v], b_v, sem1)
    cp0.wait()
    pltpu.sync_copy(s_v, sg_hbm.at[pl.ds(base, B_PER_W)])
    cp1.wait()
    pltpu.sync_copy(b_v, bg_hbm.at[pl.ds(base, B_PER_W)])


def _sc_emb_body(table_hbm, keys_hbm, emb_hbm, idx_v, rows_v, sem):
    base = _worker_base()
    pltpu.sync_copy(keys_hbm.at[pl.ds(base, B_PER_W)], idx_v)
    pltpu.async_copy(table_hbm.at[idx_v], rows_v, sem).wait()
    pltpu.sync_copy(rows_v, emb_hbm.at[pl.ds(base, B_PER_W)])


def _sc_mesh():
    return plsc.VectorSubcoreMesh(core_axis_name="c", subcore_axis_name="s",
                                  num_cores=NC, num_subcores=NS)


@functools.cache
def _sc_aux():
    return pl.kernel(
        _sc_aux_body,
        out_type=(
            jax.ShapeDtypeStruct((N_SELECTED,), jnp.float32),
            jax.ShapeDtypeStruct((N_SELECTED,), jnp.float32),
        ),
        mesh=_sc_mesh(),
        scratch_types=[
            pltpu.VMEM((B_PER_W,), jnp.int32),
            pltpu.VMEM((B_PER_W,), jnp.float32),
            pltpu.VMEM((B_PER_W,), jnp.float32),
            pltpu.VMEM_SHARED((N_NEURONS,), jnp.float32),
            pltpu.VMEM_SHARED((N_NEURONS,), jnp.float32),
            pltpu.SemaphoreType.DMA,
            pltpu.SemaphoreType.DMA,
        ],
    )


@functools.cache
def _sc_emb():
    return pl.kernel(
        _sc_emb_body,
        out_type=jax.ShapeDtypeStruct((N_SELECTED, EMBED_DIM), jnp.float32),
        mesh=_sc_mesh(),
        scratch_types=[
            pltpu.VMEM((B_PER_W,), jnp.int32),
            pltpu.VMEM((B_PER_W, EMBED_DIM), jnp.float32),
            pltpu.SemaphoreType.DMA,
        ],
    )


def _tc_affine_body(x_ref, s_ref, b_ref, o_ref):
    o_ref[...] = b_ref[...] + s_ref[...] * x_ref[...]


ROW_BLK = 512
COL_BLK = 4096

_tc_affine = pl.pallas_call(
    _tc_affine_body,
    grid=(BATCH // ROW_BLK, N_SELECTED // COL_BLK),
    in_specs=[
        pl.BlockSpec((ROW_BLK, COL_BLK), lambda i, j: (i, j)),
        pl.BlockSpec((1, COL_BLK), lambda i, j: (0, j)),
        pl.BlockSpec((1, COL_BLK), lambda i, j: (0, j)),
    ],
    out_specs=pl.BlockSpec((ROW_BLK, COL_BLK), lambda i, j: (i, j)),
    out_shape=jax.ShapeDtypeStruct((BATCH, N_SELECTED), jnp.float32),
)


def kernel(inputs, input_axon_embeddings, scale, bias, keys_idx):
    keys32 = keys_idx.astype(jnp.int32)
    sg, bg = _sc_aux()(scale, bias, keys32)
    out_emb = _sc_emb()(input_axon_embeddings, keys32)
    scale_g = sg.reshape(1, N_SELECTED)
    bias_g = bg.reshape(1, N_SELECTED)
    out_inputs = _tc_affine(inputs, scale_g, bias_g)
    return (out_inputs, out_emb)


# speedup vs baseline: 1.4173x; 1.0772x over previous
import functools

import jax
import jax.numpy as jnp
from jax import lax
from jax.experimental import pallas as pl
from jax.experimental.pallas import tpu as pltpu
from jax.experimental.pallas import tpu_sc as plsc

N_NEURONS = 1000
EMBED_DIM = 128
BATCH = 1024
N_SELECTED = 16384

NC, NS, L = 2, 16, 16
NW = NC * NS
B_PER_W = N_SELECTED // NW
AUX_F = 32
AUX_B = AUX_F * 4
HALF = B_PER_W // 2


def _worker_base():
    wid = lax.axis_index("s") * NC + lax.axis_index("c")
    return wid * B_PER_W


NSTREAM = 4
SCHUNK = B_PER_W // NSTREAM


def _sc_aux_body(scale_hbm, bias_hbm, keys_hbm, sg_hbm, bg_hbm,
                 idx_v, s_v, b_v, s_sh, b_sh, sem0, sem1):
    base = _worker_base()

    @pl.when(lax.axis_index("s") == 0)
    def _():
        pltpu.sync_copy(scale_hbm, s_sh)
        pltpu.sync_copy(bias_hbm, b_sh)

    pltpu.sync_copy(keys_hbm.at[pl.ds(base, B_PER_W)], idx_v)
    plsc.subcore_barrier()
    cp0 = pltpu.async_copy(s_sh.at[idx_v], s_v, sem0)
    cp1 = pltpu.async_copy(b_sh.at[idx_v], b_v, sem1)
    cp0.wait()
    pltpu.sync_copy(s_v, sg_hbm.at[pl.ds(base, B_PER_W)])
    cp1.wait()
    pltpu.sync_copy(b_v, bg_hbm.at[pl.ds(base, B_PER_W)])


def _sc_emb_body(table_hbm, keys_hbm, emb_hbm, idx_v, rows_v, tab_sh, sem):
    base = _worker_base()

    @pl.when(lax.axis_index("s") == 0)
    def _():
        pltpu.sync_copy(table_hbm, tab_sh)

    pltpu.sync_copy(keys_hbm.at[pl.ds(base, B_PER_W)], idx_v)
    plsc.subcore_barrier()
    pltpu.async_copy(tab_sh.at[idx_v], rows_v, sem).wait()
    pltpu.sync_copy(rows_v, emb_hbm.at[pl.ds(base, B_PER_W)])


def _sc_mesh():
    return plsc.VectorSubcoreMesh(core_axis_name="c", subcore_axis_name="s",
                                  num_cores=NC, num_subcores=NS)


@functools.cache
def _sc_aux():
    return pl.kernel(
        _sc_aux_body,
        out_type=(
            jax.ShapeDtypeStruct((N_SELECTED,), jnp.float32),
            jax.ShapeDtypeStruct((N_SELECTED,), jnp.float32),
        ),
        mesh=_sc_mesh(),
        scratch_types=[
            pltpu.VMEM((B_PER_W,), jnp.int32),
            pltpu.VMEM((B_PER_W,), jnp.float32),
            pltpu.VMEM((B_PER_W,), jnp.float32),
            pltpu.VMEM_SHARED((N_NEURONS,), jnp.float32),
            pltpu.VMEM_SHARED((N_NEURONS,), jnp.float32),
            pltpu.SemaphoreType.DMA,
            pltpu.SemaphoreType.DMA,
        ],
    )


@functools.cache
def _sc_emb():
    return pl.kernel(
        _sc_emb_body,
        out_type=jax.ShapeDtypeStruct((N_SELECTED, EMBED_DIM), jnp.float32),
        mesh=_sc_mesh(),
        scratch_types=[
            pltpu.VMEM((B_PER_W,), jnp.int32),
            pltpu.VMEM((B_PER_W, EMBED_DIM), jnp.float32),
            pltpu.VMEM_SHARED((N_NEURONS, EMBED_DIM), jnp.float32),
            pltpu.SemaphoreType.DMA,
        ],
    )


def _tc_affine_body(x_ref, s_ref, b_ref, o_ref):
    o_ref[...] = b_ref[...] + s_ref[...] * x_ref[...]


ROW_BLK = 512
COL_BLK = 4096

_tc_affine = pl.pallas_call(
    _tc_affine_body,
    grid=(BATCH // ROW_BLK, N_SELECTED // COL_BLK),
    in_specs=[
        pl.BlockSpec((ROW_BLK, COL_BLK), lambda i, j: (i, j)),
        pl.BlockSpec((1, COL_BLK), lambda i, j: (0, j)),
        pl.BlockSpec((1, COL_BLK), lambda i, j: (0, j)),
    ],
    out_specs=pl.BlockSpec((ROW_BLK, COL_BLK), lambda i, j: (i, j)),
    out_shape=jax.ShapeDtypeStruct((BATCH, N_SELECTED), jnp.float32),
)


def kernel(inputs, input_axon_embeddings, scale, bias, keys_idx):
    keys32 = keys_idx.astype(jnp.int32)
    sg, bg = _sc_aux()(scale, bias, keys32)
    out_emb = _sc_emb()(input_axon_embeddings, keys32)
    scale_g = sg.reshape(1, N_SELECTED)
    bias_g = bg.reshape(1, N_SELECTED)
    out_inputs = _tc_affine(inputs, scale_g, bias_g)
    return (out_inputs, out_emb)
